# initial kernel scaffold (unmeasured)
import jax
import jax.numpy as jnp
from jax import lax
from jax.experimental import pallas as pl
from jax.experimental.pallas import tpu as pltpu

N_DEV = 4


def kernel(x, w_mat, scale_x, scale_w):
    m_per, k = x.shape
    _, n = w_mat.shape
    n_per = n // N_DEV
    m = m_per * N_DEV

    def body(x_ref, w_ref, sx_ref, sw_ref, out_ref, acc_ref, send_sems, recv_sems):
        my = lax.axis_index("i")

        barrier = pltpu.get_barrier_semaphore()
        for d in (1, 2, 3):
            peer = lax.rem(my + d, N_DEV)
            pl.semaphore_signal(
                barrier, inc=1, device_id=(peer,),
                device_id_type=pl.DeviceIdType.MESH,
            )
        pl.semaphore_wait(barrier, N_DEV - 1)

        scale = sx_ref[0] * sw_ref[0]
        acc_ref[...] = (
            jnp.dot(
                x_ref[...].astype(jnp.bfloat16),
                w_ref[...].astype(jnp.bfloat16),
                preferred_element_type=jnp.float32,
            )
            * scale
        )

        out_ref[pl.ds(my * m_per, m_per), :] = acc_ref[:, pl.ds(my * n_per, n_per)]

        rdmas = []
        for d in (1, 2, 3):
            peer = lax.rem(my + d, N_DEV)
            rdma = pltpu.make_async_remote_copy(
                src_ref=acc_ref.at[:, pl.ds(peer * n_per, n_per)],
                dst_ref=out_ref.at[pl.ds(my * m_per, m_per), :],
                send_sem=send_sems.at[d],
                recv_sem=recv_sems.at[d],
                device_id=(peer,),
                device_id_type=pl.DeviceIdType.MESH,
            )
            rdma.start()
            rdmas.append(rdma)

        for d, rdma in zip((1, 2, 3), rdmas):
            src_peer = lax.rem(my - d + N_DEV, N_DEV)
            recv = pltpu.make_async_remote_copy(
                src_ref=acc_ref.at[:, pl.ds(src_peer * n_per, n_per)],
                dst_ref=out_ref.at[pl.ds(src_peer * m_per, m_per), :],
                send_sem=send_sems.at[d],
                recv_sem=recv_sems.at[d],
                device_id=(src_peer,),
                device_id_type=pl.DeviceIdType.MESH,
            )
            recv.wait_recv()
            rdma.wait_send()

    return pl.pallas_call(
        body,
        out_shape=jax.ShapeDtypeStruct((m, n_per), jnp.float32),
        in_specs=[
            pl.BlockSpec(memory_space=pltpu.VMEM),
            pl.BlockSpec(memory_space=pltpu.VMEM),
            pl.BlockSpec(memory_space=pltpu.SMEM),
            pl.BlockSpec(memory_space=pltpu.SMEM),
        ],
        out_specs=pl.BlockSpec(memory_space=pltpu.VMEM),
        scratch_shapes=[
            pltpu.VMEM((m_per, n), jnp.float32),
            pltpu.SemaphoreType.DMA((N_DEV,)),
            pltpu.SemaphoreType.DMA((N_DEV,)),
        ],
        compiler_params=pltpu.CompilerParams(collective_id=0),
    )(x, w_mat, scale_x, scale_w)


# baseline (device time: 64311 ns/iter reference)
import jax
import jax.numpy as jnp
from jax import lax
from jax.experimental import pallas as pl
from jax.experimental.pallas import tpu as pltpu

N_DEV = 4
KB = 1024


def _matmul(x, w_mat):
    m_per, k = x.shape
    _, n = w_mat.shape
    n_blocks = N_DEV
    n_per = n // n_blocks
    k_blocks = k // KB

    def body(x_ref, w_ref, acc_ref, xb_ref):
        kb = pl.program_id(0)
        j = pl.program_id(1)
        @pl.when(j == 0)
        def _():
            xb_ref[...] = x_ref[...].astype(jnp.bfloat16)

        partial = jnp.dot(
            xb_ref[...],
            w_ref[...].astype(jnp.bfloat16),
            preferred_element_type=jnp.float32,
        )

        @pl.when(kb == 0)
        def _():
            acc_ref[:, pl.ds(j * n_per, n_per)] = partial

        @pl.when(kb != 0)
        def _():
            acc_ref[:, pl.ds(j * n_per, n_per)] += partial

    return pl.pallas_call(
        body,
        grid=(k_blocks, n_blocks),
        in_specs=[
            pl.BlockSpec((m_per, KB), lambda kb, j: (0, kb)),
            pl.BlockSpec((KB, n_per), lambda kb, j: (kb, j)),
        ],
        out_specs=pl.BlockSpec((m_per, n), lambda kb, j: (0, 0)),
        out_shape=jax.ShapeDtypeStruct((m_per, n), jnp.float32),
        scratch_shapes=[pltpu.VMEM((m_per, KB), jnp.bfloat16)],
    )(x, w_mat)


def _a2a_dequant(acc, scale_x, scale_w):
    m_per, n = acc.shape
    n_per = n // N_DEV
    m = m_per * N_DEV

    def body(acc_ref, sx_ref, sw_ref, out_ref, sendb, recvb, send_sems, recv_sems):
        my = lax.axis_index("i")

        barrier = pltpu.get_barrier_semaphore()
        for d in (1, 2, 3):
            peer = lax.rem(my + d, N_DEV)
            pl.semaphore_signal(
                barrier, inc=1, device_id=(peer,),
                device_id_type=pl.DeviceIdType.MESH,
            )
        pl.semaphore_wait(barrier, N_DEV - 1)

        scale = sx_ref[0] * sw_ref[0]

        out_ref[pl.ds(my * m_per, m_per), :] = (
            acc_ref[:, pl.ds(my * n_per, n_per)] * scale
        )

        rdmas = []
        for d in (1, 2, 3):
            peer = lax.rem(my + d, N_DEV)
            sendb[d] = (
                acc_ref[:, pl.ds(peer * n_per, n_per)] * scale
            ).astype(jnp.bfloat16)
            rdma = pltpu.make_async_remote_copy(
                src_ref=sendb.at[d],
                dst_ref=recvb.at[d],
                send_sem=send_sems.at[d],
                recv_sem=recv_sems.at[d],
                device_id=(peer,),
                device_id_type=pl.DeviceIdType.MESH,
            )
            rdma.start()
            rdmas.append(rdma)

        for d, rdma in zip((1, 2, 3), rdmas):
            src_peer = lax.rem(my - d + N_DEV, N_DEV)
            recv = pltpu.make_async_remote_copy(
                src_ref=sendb.at[d],
                dst_ref=recvb.at[d],
                send_sem=send_sems.at[d],
                recv_sem=recv_sems.at[d],
                device_id=(src_peer,),
                device_id_type=pl.DeviceIdType.MESH,
            )
            recv.wait_recv()
            out_ref[pl.ds(src_peer * m_per, m_per), :] = recvb[d].astype(
                jnp.float32
            )
            rdma.wait_send()

    return pl.pallas_call(
        body,
        out_shape=jax.ShapeDtypeStruct((m, n_per), jnp.float32),
        in_specs=[
            pl.BlockSpec(memory_space=pltpu.VMEM),
            pl.BlockSpec(memory_space=pltpu.SMEM),
            pl.BlockSpec(memory_space=pltpu.SMEM),
        ],
        out_specs=pl.BlockSpec(memory_space=pltpu.VMEM),
        scratch_shapes=[
            pltpu.VMEM((N_DEV, m_per, n_per), jnp.bfloat16),
            pltpu.VMEM((N_DEV, m_per, n_per), jnp.bfloat16),
            pltpu.SemaphoreType.DMA((N_DEV,)),
            pltpu.SemaphoreType.DMA((N_DEV,)),
        ],
        compiler_params=pltpu.CompilerParams(collective_id=0),
    )(acc, scale_x, scale_w)


def kernel(x, w_mat, scale_x, scale_w):
    acc = _matmul(x, w_mat)
    return _a2a_dequant(acc, scale_x, scale_w)


# device time: 57169 ns/iter; 1.1249x vs baseline; 1.1249x over previous
import jax
import jax.numpy as jnp
from jax import lax
from jax.experimental import pallas as pl
from jax.experimental.pallas import tpu as pltpu

N_DEV = 4
KB = 1024


def kernel(x, w_mat, scale_x, scale_w):
    m_per, k = x.shape
    _, n = w_mat.shape
    n_per = n // N_DEV
    m = m_per * N_DEV
    k_blocks = k // KB
    d_order = (1, 2, 3, 0)
    steps = [(d, kb) for d in d_order for kb in range(k_blocks)]

    def body(x_ref, w_ref, sx_ref, sw_ref, out_ref,
             xbf_ref, wbuf, acc_ref, sendb, recvb,
             copy_sems, send_sems, recv_sems):
        my = lax.axis_index("i")

        def dest(d):
            return lax.rem(my + d, N_DEV)

        def w_copy(step, slot):
            d, kb = steps[step]
            return pltpu.make_async_copy(
                w_ref.at[pl.ds(kb * KB, KB), pl.ds(dest(d) * n_per, n_per)],
                wbuf.at[slot],
                copy_sems.at[slot],
            )

        barrier = pltpu.get_barrier_semaphore()
        for d in (1, 2, 3):
            pl.semaphore_signal(
                barrier, inc=1, device_id=(dest(d),),
                device_id_type=pl.DeviceIdType.MESH,
            )
        pl.semaphore_wait(barrier, N_DEV - 1)

        w_copy(0, 0).start()
        xbf_ref[...] = x_ref[...].astype(jnp.bfloat16)

        scale = sx_ref[0] * sw_ref[0]
        rdmas = []
        for step, (d, kb) in enumerate(steps):
            slot = step % 2
            w_copy(step, slot).wait()
            if step + 1 < len(steps):
                w_copy(step + 1, (step + 1) % 2).start()

            partial = jnp.dot(
                xbf_ref[:, pl.ds(kb * KB, KB)],
                wbuf[slot].astype(jnp.bfloat16),
                preferred_element_type=jnp.float32,
            )
            if kb == 0:
                acc_ref[...] = partial
            else:
                acc_ref[...] += partial

            if kb == k_blocks - 1:
                if d == 0:
                    out_ref[pl.ds(my * m_per, m_per), :] = acc_ref[...] * scale
                else:
                    sendb[d - 1] = (acc_ref[...] * scale).astype(jnp.bfloat16)
                    rdma = pltpu.make_async_remote_copy(
                        src_ref=sendb.at[d - 1],
                        dst_ref=recvb.at[d - 1],
                        send_sem=send_sems.at[d - 1],
                        recv_sem=recv_sems.at[d - 1],
                        device_id=(dest(d),),
                        device_id_type=pl.DeviceIdType.MESH,
                    )
                    rdma.start()
                    rdmas.append(rdma)

        for d, rdma in zip((1, 2, 3), rdmas):
            src_peer = lax.rem(my - d + N_DEV, N_DEV)
            recv = pltpu.make_async_remote_copy(
                src_ref=sendb.at[d - 1],
                dst_ref=recvb.at[d - 1],
                send_sem=send_sems.at[d - 1],
                recv_sem=recv_sems.at[d - 1],
                device_id=(src_peer,),
                device_id_type=pl.DeviceIdType.MESH,
            )
            recv.wait_recv()
            out_ref[pl.ds(src_peer * m_per, m_per), :] = recvb[d - 1].astype(
                jnp.float32
            )
            rdma.wait_send()

    return pl.pallas_call(
        body,
        out_shape=jax.ShapeDtypeStruct((m, n_per), jnp.float32),
        in_specs=[
            pl.BlockSpec(memory_space=pltpu.VMEM),
            pl.BlockSpec(memory_space=pl.MemorySpace.ANY),
            pl.BlockSpec(memory_space=pltpu.SMEM),
            pl.BlockSpec(memory_space=pltpu.SMEM),
        ],
        out_specs=pl.BlockSpec(memory_space=pltpu.VMEM),
        scratch_shapes=[
            pltpu.VMEM((m_per, k), jnp.bfloat16),
            pltpu.VMEM((2, KB, n_per), jnp.float32),
            pltpu.VMEM((m_per, n_per), jnp.float32),
            pltpu.VMEM((N_DEV - 1, m_per, n_per), jnp.bfloat16),
            pltpu.VMEM((N_DEV - 1, m_per, n_per), jnp.bfloat16),
            pltpu.SemaphoreType.DMA((2,)),
            pltpu.SemaphoreType.DMA((N_DEV - 1,)),
            pltpu.SemaphoreType.DMA((N_DEV - 1,)),
        ],
        compiler_params=pltpu.CompilerParams(
            collective_id=0,
            vmem_limit_bytes=56 * 1024 * 1024,
        ),
    )(x, w_mat, scale_x, scale_w)


# device time: 56573 ns/iter; 1.1368x vs baseline; 1.0105x over previous
import os

import jax
import jax.numpy as jnp
from jax import lax
from jax.experimental import pallas as pl
from jax.experimental.pallas import tpu as pltpu

N_DEV = 4
KB = int(os.environ.get("KERNEL_KB", "1024"))
NO_RDMA = os.environ.get("KERNEL_NO_RDMA") == "1"
NBUF = int(os.environ.get("KERNEL_NBUF", "3"))


def kernel(x, w_mat, scale_x, scale_w):
    m_per, k = x.shape
    _, n = w_mat.shape
    n_per = n // N_DEV
    m = m_per * N_DEV
    k_blocks = k // KB
    d_order = (1, 2, 3, 0)
    steps = [(d, kb) for d in d_order for kb in range(k_blocks)]
    nsteps = len(steps)

    def body(x_ref, w_ref, sx_ref, sw_ref, out_ref,
             xbf_ref, wbuf, wbf, acc_ref, sendb, recvb,
             copy_sems, send_sems, recv_sems):
        my = lax.axis_index("i")

        def dest(d):
            return lax.rem(my + d, N_DEV)

        def w_copy(step, slot):
            d, kb = steps[step]
            return pltpu.make_async_copy(
                w_ref.at[pl.ds(kb * KB, KB), pl.ds(dest(d) * n_per, n_per)],
                wbuf.at[slot],
                copy_sems.at[slot],
            )

        barrier = pltpu.get_barrier_semaphore()
        for d in (1, 2, 3):
            pl.semaphore_signal(
                barrier, inc=1, device_id=(dest(d),),
                device_id_type=pl.DeviceIdType.MESH,
            )

        w_copy(0, 0).start()
        xbf_ref[:, pl.ds(0, KB)] = x_ref[:, pl.ds(0, KB)].astype(jnp.bfloat16)
        for s in range(1, min(NBUF, nsteps)):
            w_copy(s, s % NBUF).start()
        w_copy(0, 0).wait()
        wbf[0] = wbuf[0].astype(jnp.bfloat16)

        scale = sx_ref[0] * sw_ref[0]
        rdmas = []
        for step, (d, kb) in enumerate(steps):
            if step + 1 < nsteps:
                nslot = (step + 1) % NBUF
                w_copy(step + 1, nslot).wait()
                wbf[(step + 1) % 2] = wbuf[nslot].astype(jnp.bfloat16)
            if step + 1 < k_blocks:
                c = (step + 1) * KB
                xbf_ref[:, pl.ds(c, KB)] = x_ref[:, pl.ds(c, KB)].astype(
                    jnp.bfloat16
                )

            partial = jnp.dot(
                xbf_ref[:, pl.ds(kb * KB, KB)],
                wbf[step % 2],
                preferred_element_type=jnp.float32,
            )
            if kb == 0:
                acc_ref[...] = partial
            elif kb < k_blocks - 1:
                acc_ref[...] += partial
            else:
                final = (acc_ref[...] + partial) * scale
                if d == 0:
                    out_ref[pl.ds(my * m_per, m_per), :] = final
                else:
                    sendb[d - 1] = final.astype(jnp.bfloat16)
                    if not NO_RDMA:
                        if d == d_order[0]:
                            pl.semaphore_wait(barrier, N_DEV - 1)
                        rdma = pltpu.make_async_remote_copy(
                            src_ref=sendb.at[d - 1],
                            dst_ref=recvb.at[d - 1],
                            send_sem=send_sems.at[d - 1],
                            recv_sem=recv_sems.at[d - 1],
                            device_id=(dest(d),),
                            device_id_type=pl.DeviceIdType.MESH,
                        )
                        rdma.start()
                        rdmas.append(rdma)

            if step + NBUF < nsteps:
                w_copy(step + NBUF, step % NBUF).start()

        if NO_RDMA:
            pl.semaphore_wait(barrier, N_DEV - 1)

        for d, rdma in zip((1, 2, 3), rdmas):
            src_peer = lax.rem(my - d + N_DEV, N_DEV)
            recv = pltpu.make_async_remote_copy(
                src_ref=sendb.at[d - 1],
                dst_ref=recvb.at[d - 1],
                send_sem=send_sems.at[d - 1],
                recv_sem=recv_sems.at[d - 1],
                device_id=(src_peer,),
                device_id_type=pl.DeviceIdType.MESH,
            )
            recv.wait_recv()
            out_ref[pl.ds(src_peer * m_per, m_per), :] = recvb[d - 1].astype(
                jnp.float32
            )
            rdma.wait_send()

    return pl.pallas_call(
        body,
        out_shape=jax.ShapeDtypeStruct((m, n_per), jnp.float32),
        in_specs=[
            pl.BlockSpec(memory_space=pltpu.VMEM),
            pl.BlockSpec(memory_space=pl.MemorySpace.ANY),
            pl.BlockSpec(memory_space=pltpu.SMEM),
            pl.BlockSpec(memory_space=pltpu.SMEM),
        ],
        out_specs=pl.BlockSpec(memory_space=pltpu.VMEM),
        scratch_shapes=[
            pltpu.VMEM((m_per, k), jnp.bfloat16),
            pltpu.VMEM((NBUF, KB, n_per), jnp.float32),
            pltpu.VMEM((2, KB, n_per), jnp.bfloat16),
            pltpu.VMEM((m_per, n_per), jnp.float32),
            pltpu.VMEM((N_DEV - 1, m_per, n_per), jnp.bfloat16),
            pltpu.VMEM((N_DEV - 1, m_per, n_per), jnp.bfloat16),
            pltpu.SemaphoreType.DMA((NBUF,)),
            pltpu.SemaphoreType.DMA((N_DEV - 1,)),
            pltpu.SemaphoreType.DMA((N_DEV - 1,)),
        ],
        compiler_params=pltpu.CompilerParams(
            collective_id=0,
            vmem_limit_bytes=57 * 1024 * 1024,
        ),
    )(x, w_mat, scale_x, scale_w)


# device time: 53099 ns/iter; 1.2112x vs baseline; 1.0654x over previous
import os

import jax
import jax.numpy as jnp
from jax import lax
from jax.experimental import pallas as pl
from jax.experimental.pallas import tpu as pltpu

N_DEV = 4
KB = int(os.environ.get("KERNEL_KB", "512"))
NBUF = int(os.environ.get("KERNEL_NBUF", "6"))
NO_RDMA = os.environ.get("KERNEL_NO_RDMA") == "1"
D_ORDER = (2, 1, 3, 0)


def kernel(x, w_mat, scale_x, scale_w):
    m_per, k = x.shape
    _, n = w_mat.shape
    n_per = n // N_DEV
    m = m_per * N_DEV
    k_chunks = k // KB
    steps = [(di, d, kc) for di, d in enumerate(D_ORDER) for kc in range(k_chunks)]
    nsteps = len(steps)

    def body(x_ref, w_ref, sx_ref, sw_ref, out_ref,
             xbf_ref, wbuf, wbf, sendb, recvb,
             copy_sems, send_sems, recv_sems):
        my = lax.axis_index("i")

        def dest(d):
            return lax.rem(my + d, N_DEV)

        def w_copy(step, slot):
            _, d, kc = steps[step]
            return pltpu.make_async_copy(
                w_ref.at[pl.ds(kc * KB, KB), pl.ds(dest(d) * n_per, n_per)],
                wbuf.at[slot],
                copy_sems.at[slot],
            )

        barrier = pltpu.get_barrier_semaphore()
        for d in (1, 2, 3):
            pl.semaphore_signal(
                barrier, inc=1, device_id=(dest(d),),
                device_id_type=pl.DeviceIdType.MESH,
            )

        for s in range(min(NBUF, nsteps)):
            w_copy(s, s % NBUF).start()
        xbf_ref[...] = x_ref[...].astype(jnp.bfloat16)

        scale = sx_ref[0] * sw_ref[0]
        rdmas = []
        for step, (di, d, kc) in enumerate(steps):
            slot = step % NBUF
            w_copy(step, slot).wait()
            wbf[di % 2, pl.ds(kc * KB, KB), :] = wbuf[slot].astype(jnp.bfloat16)
            if step + NBUF < nsteps:
                w_copy(step + NBUF, slot).start()

            if kc == k_chunks - 1:
                final = (
                    jnp.dot(
                        xbf_ref[...],
                        wbf[di % 2],
                        preferred_element_type=jnp.float32,
                    )
                    * scale
                )
                if d == 0:
                    out_ref[pl.ds(my * m_per, m_per), :] = final
                else:
                    sendb[d - 1] = final.astype(jnp.bfloat16)
                    if not NO_RDMA:
                        if di == 0:
                            pl.semaphore_wait(barrier, N_DEV - 1)
                        rdma = pltpu.make_async_remote_copy(
                            src_ref=sendb.at[d - 1],
                            dst_ref=recvb.at[d - 1],
                            send_sem=send_sems.at[d - 1],
                            recv_sem=recv_sems.at[d - 1],
                            device_id=(dest(d),),
                            device_id_type=pl.DeviceIdType.MESH,
                        )
                        rdma.start()
                        rdmas.append(rdma)

        if NO_RDMA:
            pl.semaphore_wait(barrier, N_DEV - 1)

        drain = [d for d in D_ORDER if d != 0]
        for d in drain:
            rdma = rdmas[drain.index(d)] if rdmas else None
            src_peer = lax.rem(my - d + N_DEV, N_DEV)
            if rdma is None:
                continue
            recv = pltpu.make_async_remote_copy(
                src_ref=sendb.at[d - 1],
                dst_ref=recvb.at[d - 1],
                send_sem=send_sems.at[d - 1],
                recv_sem=recv_sems.at[d - 1],
                device_id=(src_peer,),
                device_id_type=pl.DeviceIdType.MESH,
            )
            recv.wait_recv()
            out_ref[pl.ds(src_peer * m_per, m_per), :] = recvb[d - 1].astype(
                jnp.float32
            )
            rdma.wait_send()

    return pl.pallas_call(
        body,
        out_shape=jax.ShapeDtypeStruct((m, n_per), jnp.float32),
        in_specs=[
            pl.BlockSpec(memory_space=pltpu.VMEM),
            pl.BlockSpec(memory_space=pl.MemorySpace.ANY),
            pl.BlockSpec(memory_space=pltpu.SMEM),
            pl.BlockSpec(memory_space=pltpu.SMEM),
        ],
        out_specs=pl.BlockSpec(memory_space=pltpu.VMEM),
        scratch_shapes=[
            pltpu.VMEM((m_per, k), jnp.bfloat16),
            pltpu.VMEM((NBUF, KB, n_per), jnp.float32),
            pltpu.VMEM((2, k, n_per), jnp.bfloat16),
            pltpu.VMEM((N_DEV - 1, m_per, n_per), jnp.bfloat16),
            pltpu.VMEM((N_DEV - 1, m_per, n_per), jnp.bfloat16),
            pltpu.SemaphoreType.DMA((NBUF,)),
            pltpu.SemaphoreType.DMA((N_DEV - 1,)),
            pltpu.SemaphoreType.DMA((N_DEV - 1,)),
        ],
        compiler_params=pltpu.CompilerParams(
            collective_id=0,
            vmem_limit_bytes=60 * 1024 * 1024,
        ),
    )(x, w_mat, scale_x, scale_w)


# device time: 49282 ns/iter; 1.3050x vs baseline; 1.0775x over previous
import os

import jax
import jax.numpy as jnp
from jax import lax
from jax.experimental import pallas as pl
from jax.experimental.pallas import tpu as pltpu

N_DEV = 4
KB = int(os.environ.get("KERNEL_KB", "512"))
NBUF = int(os.environ.get("KERNEL_NBUF", "6"))
NO_RDMA = os.environ.get("KERNEL_NO_RDMA") == "1"
NO_WSTREAM = os.environ.get("KERNEL_NO_WSTREAM") == "1"
CDTYPE = jnp.float8_e5m2 if os.environ.get("KERNEL_BF16") != "1" else jnp.bfloat16
D_ORDER = (2, 1, 3, 0)


def kernel(x, w_mat, scale_x, scale_w):
    m_per, k = x.shape
    _, n = w_mat.shape
    n_per = n // N_DEV
    m = m_per * N_DEV
    k_chunks = k // KB
    steps = [(di, d, kc) for di, d in enumerate(D_ORDER) for kc in range(k_chunks)]
    nsteps = len(steps)

    def body(x_ref, w_ref, sx_ref, sw_ref, out_ref,
             xbf_ref, wbuf, wbf, sendb, recvb,
             copy_sems, send_sems, recv_sems):
        my = lax.axis_index("i")

        def dest(d):
            return lax.rem(my + d, N_DEV)

        def w_copy(step, slot):
            _, d, kc = steps[step]
            return pltpu.make_async_copy(
                w_ref.at[pl.ds(kc * KB, KB), pl.ds(dest(d) * n_per, n_per)],
                wbuf.at[slot],
                copy_sems.at[slot],
            )

        barrier = pltpu.get_barrier_semaphore()
        for d in (1, 2, 3):
            pl.semaphore_signal(
                barrier, inc=1, device_id=(dest(d),),
                device_id_type=pl.DeviceIdType.MESH,
            )

        if not NO_WSTREAM:
            for s in range(min(NBUF, nsteps)):
                w_copy(s, s % NBUF).start()
        xbf_ref[...] = x_ref[...].astype(CDTYPE)

        scale = sx_ref[0] * sw_ref[0]
        rdmas = []
        for step, (di, d, kc) in enumerate(steps):
            if not NO_WSTREAM:
                slot = step % NBUF
                w_copy(step, slot).wait()
                wbf[di % 2, pl.ds(kc * KB, KB), :] = wbuf[slot].astype(CDTYPE)
                if step + NBUF < nsteps:
                    w_copy(step + NBUF, slot).start()

            if kc == k_chunks - 1:
                final = (
                    jnp.dot(
                        xbf_ref[...],
                        wbf[di % 2],
                        preferred_element_type=jnp.float32,
                    )
                    * scale
                )
                if d == 0:
                    out_ref[pl.ds(my * m_per, m_per), :] = final
                else:
                    sendb[d - 1] = final.astype(jnp.bfloat16)
                    if not NO_RDMA:
                        if di == 0:
                            pl.semaphore_wait(barrier, N_DEV - 1)
                        rdma = pltpu.make_async_remote_copy(
                            src_ref=sendb.at[d - 1],
                            dst_ref=recvb.at[d - 1],
                            send_sem=send_sems.at[d - 1],
                            recv_sem=recv_sems.at[d - 1],
                            device_id=(dest(d),),
                            device_id_type=pl.DeviceIdType.MESH,
                        )
                        rdma.start()
                        rdmas.append(rdma)

        if NO_RDMA:
            pl.semaphore_wait(barrier, N_DEV - 1)

        drain = [d for d in D_ORDER if d != 0]
        for d in drain:
            rdma = rdmas[drain.index(d)] if rdmas else None
            src_peer = lax.rem(my - d + N_DEV, N_DEV)
            if rdma is None:
                continue
            recv = pltpu.make_async_remote_copy(
                src_ref=sendb.at[d - 1],
                dst_ref=recvb.at[d - 1],
                send_sem=send_sems.at[d - 1],
                recv_sem=recv_sems.at[d - 1],
                device_id=(src_peer,),
                device_id_type=pl.DeviceIdType.MESH,
            )
            recv.wait_recv()
            out_ref[pl.ds(src_peer * m_per, m_per), :] = recvb[d - 1].astype(
                jnp.float32
            )
            rdma.wait_send()

    return pl.pallas_call(
        body,
        out_shape=jax.ShapeDtypeStruct((m, n_per), jnp.float32),
        in_specs=[
            pl.BlockSpec(memory_space=pltpu.VMEM),
            pl.BlockSpec(memory_space=pl.MemorySpace.ANY),
            pl.BlockSpec(memory_space=pltpu.SMEM),
            pl.BlockSpec(memory_space=pltpu.SMEM),
        ],
        out_specs=pl.BlockSpec(memory_space=pltpu.VMEM),
        scratch_shapes=[
            pltpu.VMEM((m_per, k), CDTYPE),
            pltpu.VMEM((NBUF, KB, n_per), jnp.float32),
            pltpu.VMEM((2, k, n_per), CDTYPE),
            pltpu.VMEM((N_DEV - 1, m_per, n_per), jnp.bfloat16),
            pltpu.VMEM((N_DEV - 1, m_per, n_per), jnp.bfloat16),
            pltpu.SemaphoreType.DMA((NBUF,)),
            pltpu.SemaphoreType.DMA((N_DEV - 1,)),
            pltpu.SemaphoreType.DMA((N_DEV - 1,)),
        ],
        compiler_params=pltpu.CompilerParams(
            collective_id=0,
            vmem_limit_bytes=60 * 1024 * 1024,
        ),
    )(x, w_mat, scale_x, scale_w)


# device time: 49196 ns/iter; 1.3072x vs baseline; 1.0017x over previous
import os

import jax
import jax.numpy as jnp
from jax import lax
from jax.experimental import pallas as pl
from jax.experimental.pallas import tpu as pltpu

N_DEV = 4
KB = int(os.environ.get("KERNEL_KB", "512"))
NBUF = int(os.environ.get("KERNEL_NBUF", "10"))
NO_RDMA = os.environ.get("KERNEL_NO_RDMA") == "1"
NO_WSTREAM = os.environ.get("KERNEL_NO_WSTREAM") == "1"
CDTYPE = jnp.float8_e5m2 if os.environ.get("KERNEL_BF16") != "1" else jnp.bfloat16
D_ORDER = (2, 1, 3, 0)


def kernel(x, w_mat, scale_x, scale_w):
    m_per, k = x.shape
    _, n = w_mat.shape
    n_per = n // N_DEV
    m = m_per * N_DEV
    k_chunks = k // KB
    steps = [(di, d, kc) for di, d in enumerate(D_ORDER) for kc in range(k_chunks)]
    nsteps = len(steps)

    xc = m_per // 4

    def body(x_ref, w_ref, sx_ref, sw_ref, out_ref,
             xbf_ref, xstage, wbuf, wbf, sendb, recvb,
             xcopy_sems, copy_sems, send_sems, recv_sems):
        my = lax.axis_index("i")

        def dest(d):
            return lax.rem(my + d, N_DEV)

        def w_copy(step, slot):
            _, d, kc = steps[step]
            return pltpu.make_async_copy(
                w_ref.at[pl.ds(kc * KB, KB), pl.ds(dest(d) * n_per, n_per)],
                wbuf.at[slot],
                copy_sems.at[slot],
            )

        barrier = pltpu.get_barrier_semaphore()
        for d in (1, 2, 3):
            pl.semaphore_signal(
                barrier, inc=1, device_id=(dest(d),),
                device_id_type=pl.DeviceIdType.MESH,
            )

        def x_copy(c, slot):
            return pltpu.make_async_copy(
                x_ref.at[pl.ds(c * xc, xc), :],
                xstage.at[slot],
                xcopy_sems.at[slot],
            )

        if not NO_WSTREAM:
            for s in range(min(NBUF, nsteps)):
                w_copy(s, s % NBUF).start()
        x_copy(0, 0).start()
        x_copy(1, 1).start()
        for c in range(4):
            x_copy(c, c % 2).wait()
            if c + 2 < 4:
                x_copy(c + 2, c % 2).start()
            xbf_ref[pl.ds(c * xc, xc), :] = xstage[c % 2].astype(CDTYPE)

        scale = sx_ref[0] * sw_ref[0]
        rdmas = []
        for step, (di, d, kc) in enumerate(steps):
            if not NO_WSTREAM:
                slot = step % NBUF
                w_copy(step, slot).wait()
                wbf[di % 2, pl.ds(kc * KB, KB), :] = wbuf[slot].astype(CDTYPE)
                if step + NBUF < nsteps:
                    w_copy(step + NBUF, slot).start()

            if kc == k_chunks - 1:
                final = (
                    jnp.dot(
                        xbf_ref[...],
                        wbf[di % 2],
                        preferred_element_type=jnp.float32,
                    )
                    * scale
                )
                if d == 0:
                    out_ref[pl.ds(my * m_per, m_per), :] = final
                else:
                    sendb[d - 1] = final.astype(jnp.bfloat16)
                    if not NO_RDMA:
                        if di == 0:
                            pl.semaphore_wait(barrier, N_DEV - 1)
                        rdma = pltpu.make_async_remote_copy(
                            src_ref=sendb.at[d - 1],
                            dst_ref=recvb.at[d - 1],
                            send_sem=send_sems.at[d - 1],
                            recv_sem=recv_sems.at[d - 1],
                            device_id=(dest(d),),
                            device_id_type=pl.DeviceIdType.MESH,
                        )
                        rdma.start()
                        rdmas.append(rdma)

        if NO_RDMA:
            pl.semaphore_wait(barrier, N_DEV - 1)

        drain = [d for d in D_ORDER if d != 0]
        for d in drain:
            rdma = rdmas[drain.index(d)] if rdmas else None
            src_peer = lax.rem(my - d + N_DEV, N_DEV)
            if rdma is None:
                continue
            recv = pltpu.make_async_remote_copy(
                src_ref=sendb.at[d - 1],
                dst_ref=recvb.at[d - 1],
                send_sem=send_sems.at[d - 1],
                recv_sem=recv_sems.at[d - 1],
                device_id=(src_peer,),
                device_id_type=pl.DeviceIdType.MESH,
            )
            recv.wait_recv()
            out_ref[pl.ds(src_peer * m_per, m_per), :] = recvb[d - 1].astype(
                jnp.float32
            )
            rdma.wait_send()

    return pl.pallas_call(
        body,
        out_shape=jax.ShapeDtypeStruct((m, n_per), jnp.float32),
        in_specs=[
            pl.BlockSpec(memory_space=pl.MemorySpace.ANY),
            pl.BlockSpec(memory_space=pl.MemorySpace.ANY),
            pl.BlockSpec(memory_space=pltpu.SMEM),
            pl.BlockSpec(memory_space=pltpu.SMEM),
        ],
        out_specs=pl.BlockSpec(memory_space=pltpu.VMEM),
        scratch_shapes=[
            pltpu.VMEM((m_per, k), CDTYPE),
            pltpu.VMEM((2, m_per // 4, k), jnp.float32),
            pltpu.VMEM((NBUF, KB, n_per), jnp.float32),
            pltpu.VMEM((2, k, n_per), CDTYPE),
            pltpu.VMEM((N_DEV - 1, m_per, n_per), jnp.bfloat16),
            pltpu.VMEM((N_DEV - 1, m_per, n_per), jnp.bfloat16),
            pltpu.SemaphoreType.DMA((2,)),
            pltpu.SemaphoreType.DMA((NBUF,)),
            pltpu.SemaphoreType.DMA((N_DEV - 1,)),
            pltpu.SemaphoreType.DMA((N_DEV - 1,)),
        ],
        compiler_params=pltpu.CompilerParams(
            collective_id=0,
            vmem_limit_bytes=60 * 1024 * 1024,
        ),
    )(x, w_mat, scale_x, scale_w)
